# baseline (speedup 1.0000x reference)
import functools

import jax
import jax.numpy as jnp
from jax import lax
from jax.experimental import pallas as pl
from jax.experimental.pallas import tpu as pltpu
from jax.experimental.pallas import tpu_sc as plsc

_LQ = 1024
_LK = 1024
_D = 256
_ROWS = 9
_MAXK = 4
_T = 4096
_MID = _T // 2 - 2

_NC = 2
_NS = 16
_ROWS_PER_TILE = _LQ // (_NC * _NS)


def _build_body(off_ref, pe_ref, t_ref):
    u = jax.lax.broadcasted_iota(jnp.int32, (_T, _D), 0)
    c = jnp.clip(u - _MID + off_ref[0], -_MAXK, _MAXK) + _MAXK
    acc = jnp.zeros((_T, _D), jnp.float32)
    for r in range(_ROWS):
        acc = jnp.where(c == r, pe_ref[r, :][None, :], acc)
    t_ref[...] = acc


def _build_template(off, pe):
    return pl.pallas_call(
        _build_body,
        in_specs=[
            pl.BlockSpec(memory_space=pltpu.SMEM),
            pl.BlockSpec((_ROWS, _D), lambda: (0, 0)),
        ],
        out_specs=pl.BlockSpec((_T, _D), lambda: (0, 0)),
        out_shape=jax.ShapeDtypeStruct((_T, _D), jnp.float32),
    )(off, pe)


_ROW_W = _LK * _D


def _sc_copy_body(t_hbm, out_hbm, shared, sem):
    cid = lax.axis_index("c")
    sid = lax.axis_index("s")
    wid = cid * _NS + sid

    @pl.when(sid == 0)
    def _load():
        pltpu.sync_copy(t_hbm, shared)

    plsc.subcore_barrier()

    def _row(j, carry):
        q = wid * _ROWS_PER_TILE + j
        start = (_MID - q) * _D
        pltpu.async_copy(
            shared.at[pl.ds(start, _ROW_W)],
            out_hbm.at[pl.ds(q * _ROW_W, _ROW_W)],
            sem,
        ).wait()
        return carry

    lax.fori_loop(0, _ROWS_PER_TILE, _row, 0)


_sc_copy = pl.kernel(
    _sc_copy_body,
    out_type=jax.ShapeDtypeStruct((_LQ * _LK * _D,), jnp.float32),
    mesh=plsc.VectorSubcoreMesh(
        core_axis_name="c", subcore_axis_name="s",
        num_cores=_NC, num_subcores=_NS,
    ),
    scratch_types=[
        pltpu.VMEM_SHARED((_T * _D,), jnp.float32),
        pltpu.SemaphoreType.DMA,
    ],
)


def kernel(length_q, length_k, pe):
    off = jnp.asarray(length_k - length_q, jnp.int32).reshape((1,))
    t = _build_template(off, pe).reshape((_T * _D,))
    return _sc_copy(t).reshape((_LQ, _LK, _D))


# speedup vs baseline: 1.0058x; 1.0058x over previous
import functools

import jax
import jax.numpy as jnp
from jax import lax
from jax.experimental import pallas as pl
from jax.experimental.pallas import tpu as pltpu
from jax.experimental.pallas import tpu_sc as plsc

_LQ = 1024
_LK = 1024
_D = 256
_ROWS = 9
_MAXK = 4
_T = 4096
_MID = _T // 2 - 2

_NC = 2
_NS = 16
_ROWS_PER_TILE = _LQ // (_NC * _NS)


def _build_body(off_ref, pe_ref, t_ref):
    u = jax.lax.broadcasted_iota(jnp.int32, (_T, _D), 0)
    c = jnp.clip(u - _MID + off_ref[0], -_MAXK, _MAXK) + _MAXK
    acc = jnp.zeros((_T, _D), jnp.float32)
    for r in range(_ROWS):
        acc = jnp.where(c == r, pe_ref[r, :][None, :], acc)
    t_ref[...] = acc


def _build_template(off, pe):
    return pl.pallas_call(
        _build_body,
        in_specs=[
            pl.BlockSpec(memory_space=pltpu.SMEM),
            pl.BlockSpec((_ROWS, _D), lambda: (0, 0)),
        ],
        out_specs=pl.BlockSpec((_T, _D), lambda: (0, 0)),
        out_shape=jax.ShapeDtypeStruct((_T, _D), jnp.float32),
    )(off, pe)


_ROW_W = _LK * _D


def _sc_copy_body(t_hbm, out_hbm, shared, sem):
    cid = lax.axis_index("c")
    sid = lax.axis_index("s")
    wid = cid * _NS + sid

    @pl.when(sid == 0)
    def _load():
        pltpu.sync_copy(t_hbm, shared)

    plsc.subcore_barrier()

    def _row(j, carry):
        q = wid * _ROWS_PER_TILE + j
        start = (_MID - q) * _D
        pltpu.async_copy(
            shared.at[pl.ds(start, _ROW_W)],
            out_hbm.at[pl.ds(q * _ROW_W, _ROW_W)],
            sem,
        )
        return carry

    lax.fori_loop(0, _ROWS_PER_TILE, _row, 0)

    def _drain(j, carry):
        pltpu.make_async_copy(
            shared.at[pl.ds(0, _ROW_W)],
            out_hbm.at[pl.ds(0, _ROW_W)],
            sem,
        ).wait()
        return carry

    lax.fori_loop(0, _ROWS_PER_TILE, _drain, 0)


_sc_copy = pl.kernel(
    _sc_copy_body,
    out_type=jax.ShapeDtypeStruct((_LQ * _LK * _D,), jnp.float32),
    mesh=plsc.VectorSubcoreMesh(
        core_axis_name="c", subcore_axis_name="s",
        num_cores=_NC, num_subcores=_NS,
    ),
    scratch_types=[
        pltpu.VMEM_SHARED((_T * _D,), jnp.float32),
        pltpu.SemaphoreType.DMA,
    ],
)


def kernel(length_q, length_k, pe):
    off = jnp.asarray(length_k - length_q, jnp.int32).reshape((1,))
    t = _build_template(off, pe).reshape((_T * _D,))
    return _sc_copy(t).reshape((_LQ, _LK, _D))


# speedup vs baseline: 1.0090x; 1.0031x over previous
import functools

import jax
import jax.numpy as jnp
from jax import lax
from jax.experimental import pallas as pl
from jax.experimental.pallas import tpu as pltpu
from jax.experimental.pallas import tpu_sc as plsc

_LQ = 1024
_LK = 1024
_D = 256
_ROWS = 9
_MAXK = 4
_T = 4096
_MID = _T // 2 - 2

_NC = 2
_NS = 16
_ROWS_PER_TILE = _LQ // (_NC * _NS)


def _build_body(off_ref, pe_ref, t_ref):
    u = jax.lax.broadcasted_iota(jnp.int32, (_T, _D), 0)
    c = jnp.clip(u - _MID + off_ref[0], -_MAXK, _MAXK) + _MAXK
    acc = jnp.zeros((_T, _D), jnp.float32)
    for r in range(_ROWS):
        acc = jnp.where(c == r, pe_ref[r, :][None, :], acc)
    t_ref[...] = acc


def _build_template(off, pe):
    return pl.pallas_call(
        _build_body,
        in_specs=[
            pl.BlockSpec(memory_space=pltpu.SMEM),
            pl.BlockSpec((_ROWS, _D), lambda: (0, 0)),
        ],
        out_specs=pl.BlockSpec((_T, _D), lambda: (0, 0)),
        out_shape=jax.ShapeDtypeStruct((_T, _D), jnp.float32),
    )(off, pe)


def _sc_copy_body(t_hbm, out_hbm, shared, sem):
    cid = lax.axis_index("c")
    sid = lax.axis_index("s")
    wid = cid * _NS + sid

    @pl.when(sid == 0)
    def _load():
        pltpu.sync_copy(t_hbm, shared)

    plsc.subcore_barrier()

    def _row(j, carry):
        q = wid * _ROWS_PER_TILE + j
        start = _MID - q
        pltpu.async_copy(
            shared.at[pl.ds(start, _LK)],
            out_hbm.at[q],
            sem,
        )
        return carry

    lax.fori_loop(0, _ROWS_PER_TILE, _row, 0)

    def _drain(j, carry):
        pltpu.make_async_copy(
            shared.at[pl.ds(0, _LK)],
            out_hbm.at[0],
            sem,
        ).wait()
        return carry

    lax.fori_loop(0, _ROWS_PER_TILE, _drain, 0)


_sc_copy = pl.kernel(
    _sc_copy_body,
    out_type=jax.ShapeDtypeStruct((_LQ, _LK, _D), jnp.float32),
    mesh=plsc.VectorSubcoreMesh(
        core_axis_name="c", subcore_axis_name="s",
        num_cores=_NC, num_subcores=_NS,
    ),
    scratch_types=[
        pltpu.VMEM_SHARED((_T, _D), jnp.float32),
        pltpu.SemaphoreType.DMA,
    ],
    compiler_params=pltpu.CompilerParams(use_tc_tiling_on_sc=False),
)


def kernel(length_q, length_k, pe):
    off = jnp.asarray(length_k - length_q, jnp.int32).reshape((1,))
    t = _build_template(off, pe)
    return _sc_copy(t)
